# baseline (speedup 1.0000x reference)
import functools

import jax
import jax.numpy as jnp
from jax import lax
from jax.experimental import pallas as pl
from jax.experimental.pallas import tpu as pltpu
from jax.experimental.pallas import tpu_sc as plsc

NUM_HEADS = 12
HIDDEN_DIM = 768
MLP_DIM = 3072
N_HASHES = 2
Q_CLUSTER = 128
K_CLUSTER = 128
HEAD_DIM = HIDDEN_DIM // NUM_HEADS
S = 4096
NPART = 3 * NUM_HEADS

ROW_BLK = 512
ATTN_BLK = 8
MLP_BLK = 768


def _ln(x, w, b, eps=1e-6):
    mu = jnp.mean(x, axis=-1, keepdims=True)
    var = jnp.mean((x - mu) ** 2, axis=-1, keepdims=True)
    return (x - mu) / jnp.sqrt(var + eps) * w + b


def _bdot(a, b, dims):
    return jax.lax.dot_general(a.astype(jnp.bfloat16), b.astype(jnp.bfloat16),
                               dims, preferred_element_type=jnp.float32)


def _k1_body(x_ref, w_ref, lw_ref, lb_ref, bq_ref, out_ref):
    xn = _ln(x_ref[...], lw_ref[...], lb_ref[...])
    acc = _bdot(xn, w_ref[...], (((1,), (1,)), ((), ())))
    col = jax.lax.broadcasted_iota(jnp.int32, (1, 3 * HIDDEN_DIM), 1)
    scale = jnp.where(col < HIDDEN_DIM, HEAD_DIM ** -0.5, 1.0)
    out_ref[...] = acc * scale + bq_ref[...] * scale


def _ln_qkv(x, ln1_w, ln1_b, w_qkv, b_qkv):
    return pl.pallas_call(
        _k1_body,
        grid=(S // ROW_BLK,),
        in_specs=[
            pl.BlockSpec((ROW_BLK, HIDDEN_DIM), lambda r: (r, 0)),
            pl.BlockSpec((3 * HIDDEN_DIM, HIDDEN_DIM), lambda r: (0, 0)),
            pl.BlockSpec((1, HIDDEN_DIM), lambda r: (0, 0)),
            pl.BlockSpec((1, HIDDEN_DIM), lambda r: (0, 0)),
            pl.BlockSpec((1, 3 * HIDDEN_DIM), lambda r: (0, 0)),
        ],
        out_specs=pl.BlockSpec((ROW_BLK, 3 * HIDDEN_DIM), lambda r: (r, 0)),
        out_shape=jax.ShapeDtypeStruct((S, 3 * HIDDEN_DIM), jnp.float32),
    )(x, w_qkv, ln1_w.reshape(1, -1), ln1_b.reshape(1, -1), b_qkv.reshape(1, -1))


def _k2_body(q_ref, kv_ref, bo_ref, lse_ref):
    for i in range(ATTN_BLK):
        q = q_ref[i]
        kv = kv_ref[i]
        k = kv[:, :HEAD_DIM]
        v = kv[:, HEAD_DIM:]
        inner = _bdot(q, k, (((1,), (1,)), ((), ())))
        m = jnp.max(inner, axis=-1, keepdims=True)
        e = jnp.exp(inner - m)
        s = jnp.sum(e, axis=-1, keepdims=True)
        bo = _bdot(e / s, v, (((1,), (0,)), ((), ())))
        bo_ref[i] = bo
        lse_ref[i] = (m + jnp.log(s))[:, 0]


def _cluster_attn(s_q, s_kv):
    n_c = s_q.shape[0]
    return pl.pallas_call(
        _k2_body,
        grid=(n_c // ATTN_BLK,),
        in_specs=[
            pl.BlockSpec((ATTN_BLK, Q_CLUSTER, HEAD_DIM), lambda c: (c, 0, 0)),
            pl.BlockSpec((ATTN_BLK, K_CLUSTER, 2 * HEAD_DIM), lambda c: (c, 0, 0)),
        ],
        out_specs=[
            pl.BlockSpec((ATTN_BLK, Q_CLUSTER, HEAD_DIM), lambda c: (c, 0, 0)),
            pl.BlockSpec((ATTN_BLK, Q_CLUSTER), lambda c: (c, 0)),
        ],
        out_shape=[
            jax.ShapeDtypeStruct((n_c, Q_CLUSTER, HEAD_DIM), jnp.float32),
            jax.ShapeDtypeStruct((n_c, Q_CLUSTER), jnp.float32),
        ],
    )(s_q, s_kv)


def _k3_body(o1_ref, o2_ref, p_ref, x_ref, wo_ref, bo_ref, l2w_ref, l2b_ref,
             w1_ref, b1_ref, w2_ref, b2_ref, out_ref, h_s, y_s):
    m = pl.program_id(1)

    @pl.when(m == 0)
    def _():
        p = p_ref[...]
        row = jax.lax.broadcasted_iota(jnp.int32, (NUM_HEADS, HIDDEN_DIM), 0)
        col = jax.lax.broadcasted_iota(jnp.int32, (NUM_HEADS, HIDDEN_DIM), 1)
        E = jnp.where(col // HEAD_DIM == row, 1.0, 0.0)
        pe1 = jax.lax.dot_general(p[:, :NUM_HEADS], E, (((1,), (0,)), ((), ())),
                                  preferred_element_type=jnp.float32)
        pe2 = jax.lax.dot_general(p[:, NUM_HEADS:], E, (((1,), (0,)), ((), ())),
                                  preferred_element_type=jnp.float32)
        attn = o1_ref[...] * pe1 + o2_ref[...] * pe2
        h = _bdot(attn, wo_ref[...], (((1,), (1,)), ((), ())))
        h = h + bo_ref[...] + x_ref[...]
        h_s[...] = h
        y_s[...] = _ln(h, l2w_ref[...], l2b_ref[...])

    z = _bdot(y_s[...], w1_ref[...], (((1,), (1,)), ((), ())))
    z = z + b1_ref[...]
    z = 0.5 * z * (1.0 + jax.lax.erf(z * (2.0 ** -0.5)))
    part = _bdot(z, w2_ref[...], (((1,), (1,)), ((), ())))

    @pl.when(m == 0)
    def _():
        out_ref[...] = h_s[...] + b2_ref[...] + part

    @pl.when(m != 0)
    def _():
        out_ref[...] = out_ref[...] + part


def _fused_tail(o_tm1, o_tm2, probs_tm, x, w_out, b_out, ln2_w, ln2_b,
                w1, b1, w2, b2):
    n_m = MLP_DIM // MLP_BLK
    return pl.pallas_call(
        _k3_body,
        grid=(S // ROW_BLK, n_m),
        in_specs=[
            pl.BlockSpec((ROW_BLK, HIDDEN_DIM), lambda r, m: (r, 0)),
            pl.BlockSpec((ROW_BLK, HIDDEN_DIM), lambda r, m: (r, 0)),
            pl.BlockSpec((ROW_BLK, N_HASHES * NUM_HEADS), lambda r, m: (r, 0)),
            pl.BlockSpec((ROW_BLK, HIDDEN_DIM), lambda r, m: (r, 0)),
            pl.BlockSpec((HIDDEN_DIM, HIDDEN_DIM), lambda r, m: (0, 0)),
            pl.BlockSpec((1, HIDDEN_DIM), lambda r, m: (0, 0)),
            pl.BlockSpec((1, HIDDEN_DIM), lambda r, m: (0, 0)),
            pl.BlockSpec((1, HIDDEN_DIM), lambda r, m: (0, 0)),
            pl.BlockSpec((MLP_BLK, HIDDEN_DIM), lambda r, m: (m, 0)),
            pl.BlockSpec((1, MLP_BLK), lambda r, m: (0, m)),
            pl.BlockSpec((HIDDEN_DIM, MLP_BLK), lambda r, m: (0, m)),
            pl.BlockSpec((1, HIDDEN_DIM), lambda r, m: (0, 0)),
        ],
        out_specs=pl.BlockSpec((ROW_BLK, HIDDEN_DIM), lambda r, m: (r, 0)),
        out_shape=jax.ShapeDtypeStruct((S, HIDDEN_DIM), jnp.float32),
        scratch_shapes=[
            pltpu.VMEM((ROW_BLK, HIDDEN_DIM), jnp.float32),
            pltpu.VMEM((ROW_BLK, HIDDEN_DIM), jnp.float32),
        ],
    )(o_tm1, o_tm2, probs_tm, x, w_out, b_out.reshape(1, -1), ln2_w.reshape(1, -1),
      ln2_b.reshape(1, -1), w1, b1.reshape(1, -1), w2, b2.reshape(1, -1))


_N_PERM = N_HASHES * NUM_HEADS


def _sc_inv_body(pos_hbm, out_hbm, pos_v, tab_v):
    wid = lax.axis_index("s") * 2 + lax.axis_index("c")

    @pl.when(wid < _N_PERM)
    def _():
        pltpu.sync_copy(pos_hbm.at[wid], pos_v)

        def body(j, carry):
            idx = pos_v[pl.ds(j * 16, 16)]
            val = lax.iota(jnp.int32, 16) + j * 16
            plsc.store_scatter(tab_v, [idx], val)
            return carry

        lax.fori_loop(0, S // 16, body, 0)
        pltpu.sync_copy(tab_v, out_hbm.at[wid])


def _sc_invert_perm(pos):
    fn = functools.partial(
        pl.kernel,
        mesh=plsc.VectorSubcoreMesh(core_axis_name="c", subcore_axis_name="s"),
        out_type=jax.ShapeDtypeStruct((_N_PERM, S), jnp.int32),
        scratch_types=[
            pltpu.VMEM((S,), jnp.int32),
            pltpu.VMEM((S,), jnp.int32),
        ],
        compiler_params=pltpu.CompilerParams(needs_layout_passes=False),
    )(_sc_inv_body)
    return fn(pos)


def kernel(x, ln1_w, ln1_b, w_qkv, b_qkv, w_out, b_out, ln2_w, ln2_b,
           w1, b1, w2, b2, alpha, beta):
    x2 = x[0]
    hd_ids = jnp.arange(NUM_HEADS, dtype=jnp.int32)
    kv_parts = jnp.stack([NUM_HEADS + hd_ids, 2 * NUM_HEADS + hd_ids],
                         axis=1).reshape(-1)
    parts = jnp.concatenate([hd_ids, kv_parts])
    perm = (parts[:, None] * HEAD_DIM
            + jnp.arange(HEAD_DIM, dtype=jnp.int32)[None, :]).reshape(-1)
    qkv = _ln_qkv(x2, ln1_w, ln1_b, w_qkv[perm], b_qkv[perm])
    qkv_flat = qkv.reshape(S * NPART, HEAD_DIM)
    qkv_flat128 = qkv.reshape(S * NPART // 2, 2 * HEAD_DIM)

    def heads_cols(lo):
        return qkv[:, lo:lo + HIDDEN_DIM].reshape(
            S, NUM_HEADS, HEAD_DIM).transpose(1, 0, 2)

    q = heads_cols(0)
    k = qkv[:, HIDDEN_DIM:].reshape(S, NUM_HEADS, 2, HEAD_DIM)[:, :, 0, :]
    k = k.transpose(1, 0, 2)

    q_norms = jnp.linalg.norm(q, axis=-1, keepdims=True)
    k_norms = jnp.linalg.norm(k, axis=-1, keepdims=True)
    MX = jnp.max(q_norms, axis=1, keepdims=True)
    MY = jnp.max(k_norms, axis=1, keepdims=True)
    q_ext = jnp.sqrt(jnp.maximum(MX ** 2 + MY ** 2 - q_norms ** 2, 0.0))
    k_ext = jnp.sqrt(jnp.maximum(MX ** 2 + MY ** 2 - k_norms ** 2, 0.0))
    Qh = jnp.concatenate([q, q_ext, jnp.zeros_like(q_ext)], axis=-1)
    Kh = jnp.concatenate([k, jnp.zeros_like(k_ext), k_ext], axis=-1)
    q_proj = Qh @ alpha + beta
    k_proj = Kh @ alpha + beta

    keys = jnp.stack([q_proj.transpose(2, 0, 1), k_proj.transpose(2, 0, 1)])
    kb = jax.lax.bitcast_convert_type(keys, jnp.int32)
    kb = kb ^ ((kb >> 31) & jnp.int32(0x7FFFFFFF))
    positions = jnp.argsort(kb, axis=-1)
    q_positions, k_positions = positions[0], positions[1]

    iota_s = jnp.broadcast_to(jnp.arange(S, dtype=jnp.int32), q_positions.shape)
    q_rev = _sc_invert_perm(q_positions.reshape(N_HASHES * NUM_HEADS, S))
    q_rev = q_rev.reshape(q_positions.shape)

    head_part = jnp.arange(NUM_HEADS, dtype=jnp.int32)[:, None]
    g = jnp.arange(NUM_HEADS, dtype=jnp.int32)
    o_tm = []
    slog = []
    for h2 in range(N_HASHES):
        q_idx = q_positions[h2] * NPART + head_part
        kv_idx = k_positions[h2] * (NPART // 2) + NUM_HEADS // 2 + head_part
        s_q = jnp.take(qkv_flat, q_idx.reshape(-1), axis=0).reshape(
            -1, Q_CLUSTER, HEAD_DIM)
        s_kv = jnp.take(qkv_flat128, kv_idx.reshape(-1), axis=0).reshape(
            -1, K_CLUSTER, 2 * HEAD_DIM)
        bo, lse = _cluster_attn(s_q, s_kv)
        o_idx = g[None, :] * S + q_rev[h2].reshape(NUM_HEADS, S).T
        o_tm.append(jnp.take(bo.reshape(-1, HEAD_DIM), o_idx.reshape(-1),
                             axis=0).reshape(S, HIDDEN_DIM))
        slog.append(jnp.take(lse.reshape(-1), o_idx.reshape(-1),
                             axis=0).reshape(S, 1, NUM_HEADS))
    slog = jnp.concatenate(slog, axis=1)
    mx = jnp.max(slog, axis=1, keepdims=True)
    w = jnp.exp(slog - mx)
    probs_tm = (w / jnp.sum(w, axis=1, keepdims=True)).reshape(
        S, N_HASHES * NUM_HEADS)

    out = _fused_tail(o_tm[0], o_tm[1], probs_tm, x2, w_out, b_out, ln2_w,
                      ln2_b, w1, b1, w2, b2)
    return out[None]


# speedup vs baseline: 1.0478x; 1.0478x over previous
import functools

import jax
import jax.numpy as jnp
from jax import lax
from jax.experimental import pallas as pl
from jax.experimental.pallas import tpu as pltpu
from jax.experimental.pallas import tpu_sc as plsc

NUM_HEADS = 12
HIDDEN_DIM = 768
MLP_DIM = 3072
N_HASHES = 2
Q_CLUSTER = 128
K_CLUSTER = 128
HEAD_DIM = HIDDEN_DIM // NUM_HEADS
S = 4096
NPART = 3 * NUM_HEADS

ROW_BLK = 512
ATTN_BLK = 8
MLP_BLK = 768


def _ln(x, w, b, eps=1e-6):
    mu = jnp.mean(x, axis=-1, keepdims=True)
    var = jnp.mean((x - mu) ** 2, axis=-1, keepdims=True)
    return (x - mu) / jnp.sqrt(var + eps) * w + b


def _bdot(a, b, dims):
    return jax.lax.dot_general(a.astype(jnp.bfloat16), b.astype(jnp.bfloat16),
                               dims, preferred_element_type=jnp.float32)


def _k1_body(x_ref, w_ref, lw_ref, lb_ref, bq_ref, out_ref):
    xn = _ln(x_ref[...], lw_ref[...], lb_ref[...])
    acc = _bdot(xn, w_ref[...], (((1,), (1,)), ((), ())))
    col = jax.lax.broadcasted_iota(jnp.int32, (1, 3 * HIDDEN_DIM), 1)
    scale = jnp.where(col < HIDDEN_DIM, HEAD_DIM ** -0.5, 1.0)
    out_ref[...] = acc * scale + bq_ref[...] * scale


def _ln_qkv(x, ln1_w, ln1_b, w_qkv, b_qkv):
    return pl.pallas_call(
        _k1_body,
        grid=(S // ROW_BLK,),
        in_specs=[
            pl.BlockSpec((ROW_BLK, HIDDEN_DIM), lambda r: (r, 0)),
            pl.BlockSpec((3 * HIDDEN_DIM, HIDDEN_DIM), lambda r: (0, 0)),
            pl.BlockSpec((1, HIDDEN_DIM), lambda r: (0, 0)),
            pl.BlockSpec((1, HIDDEN_DIM), lambda r: (0, 0)),
            pl.BlockSpec((1, 3 * HIDDEN_DIM), lambda r: (0, 0)),
        ],
        out_specs=pl.BlockSpec((ROW_BLK, 3 * HIDDEN_DIM), lambda r: (r, 0)),
        out_shape=jax.ShapeDtypeStruct((S, 3 * HIDDEN_DIM), jnp.float32),
    )(x, w_qkv, ln1_w.reshape(1, -1), ln1_b.reshape(1, -1), b_qkv.reshape(1, -1))


def _k2_body(q_ref, kv_ref, bo_ref, lse_ref):
    for i in range(ATTN_BLK):
        q = q_ref[i]
        kv = kv_ref[i]
        k = kv[:, :HEAD_DIM]
        v = kv[:, HEAD_DIM:]
        inner = _bdot(q, k, (((1,), (1,)), ((), ())))
        m = jnp.max(inner, axis=-1, keepdims=True)
        e = jnp.exp(inner - m)
        s = jnp.sum(e, axis=-1, keepdims=True)
        bo = _bdot(e / s, v, (((1,), (0,)), ((), ())))
        bo_ref[i] = bo
        lse_ref[i] = (m + jnp.log(s))[:, 0]


def _cluster_attn(s_q, s_kv):
    n_c = s_q.shape[0]
    return pl.pallas_call(
        _k2_body,
        grid=(n_c // ATTN_BLK,),
        in_specs=[
            pl.BlockSpec((ATTN_BLK, Q_CLUSTER, HEAD_DIM), lambda c: (c, 0, 0)),
            pl.BlockSpec((ATTN_BLK, K_CLUSTER, 2 * HEAD_DIM), lambda c: (c, 0, 0)),
        ],
        out_specs=[
            pl.BlockSpec((ATTN_BLK, Q_CLUSTER, HEAD_DIM), lambda c: (c, 0, 0)),
            pl.BlockSpec((ATTN_BLK, Q_CLUSTER), lambda c: (c, 0)),
        ],
        out_shape=[
            jax.ShapeDtypeStruct((n_c, Q_CLUSTER, HEAD_DIM), jnp.float32),
            jax.ShapeDtypeStruct((n_c, Q_CLUSTER), jnp.float32),
        ],
    )(s_q, s_kv)


def _k3_body(o_ref, p_ref, x_ref, wo_ref, bo_ref, l2w_ref, l2b_ref,
             w1_ref, b1_ref, w2_ref, b2_ref, out_ref, h_s, y_s):
    m = pl.program_id(1)

    @pl.when(m == 0)
    def _():
        o = o_ref[...]
        p = p_ref[...]
        row = jax.lax.broadcasted_iota(jnp.int32, (NUM_HEADS, HIDDEN_DIM), 0)
        col = jax.lax.broadcasted_iota(jnp.int32, (NUM_HEADS, HIDDEN_DIM), 1)
        E = jnp.where(col // HEAD_DIM == row, 1.0, 0.0)
        pe1 = jax.lax.dot_general(p[:, :NUM_HEADS], E, (((1,), (0,)), ((), ())),
                                  preferred_element_type=jnp.float32)
        pe2 = jax.lax.dot_general(p[:, NUM_HEADS:], E, (((1,), (0,)), ((), ())),
                                  preferred_element_type=jnp.float32)
        attn = o[:, :HIDDEN_DIM] * pe1 + o[:, HIDDEN_DIM:] * pe2
        h = _bdot(attn, wo_ref[...], (((1,), (1,)), ((), ())))
        h = h + bo_ref[...] + x_ref[...]
        h_s[...] = h
        y_s[...] = _ln(h, l2w_ref[...], l2b_ref[...])

    z = _bdot(y_s[...], w1_ref[...], (((1,), (1,)), ((), ())))
    z = z + b1_ref[...]
    z = 0.5 * z * (1.0 + jax.lax.erf(z * (2.0 ** -0.5)))
    part = _bdot(z, w2_ref[...], (((1,), (1,)), ((), ())))

    @pl.when(m == 0)
    def _():
        out_ref[...] = h_s[...] + b2_ref[...] + part

    @pl.when(m != 0)
    def _():
        out_ref[...] = out_ref[...] + part


def _fused_tail(o_tm, probs_tm, x, w_out, b_out, ln2_w, ln2_b, w1, b1, w2, b2):
    n_m = MLP_DIM // MLP_BLK
    return pl.pallas_call(
        _k3_body,
        grid=(S // ROW_BLK, n_m),
        in_specs=[
            pl.BlockSpec((ROW_BLK, N_HASHES * HIDDEN_DIM), lambda r, m: (r, 0)),
            pl.BlockSpec((ROW_BLK, N_HASHES * NUM_HEADS), lambda r, m: (r, 0)),
            pl.BlockSpec((ROW_BLK, HIDDEN_DIM), lambda r, m: (r, 0)),
            pl.BlockSpec((HIDDEN_DIM, HIDDEN_DIM), lambda r, m: (0, 0)),
            pl.BlockSpec((1, HIDDEN_DIM), lambda r, m: (0, 0)),
            pl.BlockSpec((1, HIDDEN_DIM), lambda r, m: (0, 0)),
            pl.BlockSpec((1, HIDDEN_DIM), lambda r, m: (0, 0)),
            pl.BlockSpec((MLP_BLK, HIDDEN_DIM), lambda r, m: (m, 0)),
            pl.BlockSpec((1, MLP_BLK), lambda r, m: (0, m)),
            pl.BlockSpec((HIDDEN_DIM, MLP_BLK), lambda r, m: (0, m)),
            pl.BlockSpec((1, HIDDEN_DIM), lambda r, m: (0, 0)),
        ],
        out_specs=pl.BlockSpec((ROW_BLK, HIDDEN_DIM), lambda r, m: (r, 0)),
        out_shape=jax.ShapeDtypeStruct((S, HIDDEN_DIM), jnp.float32),
        scratch_shapes=[
            pltpu.VMEM((ROW_BLK, HIDDEN_DIM), jnp.float32),
            pltpu.VMEM((ROW_BLK, HIDDEN_DIM), jnp.float32),
        ],
    )(o_tm, probs_tm, x, w_out, b_out.reshape(1, -1), ln2_w.reshape(1, -1),
      ln2_b.reshape(1, -1), w1, b1.reshape(1, -1), w2, b2.reshape(1, -1))


_N_PERM = N_HASHES * NUM_HEADS


def _sc_inv_body(pos_hbm, out_hbm, pos_v, tab_v):
    wid = lax.axis_index("s") * 2 + lax.axis_index("c")

    @pl.when(wid < _N_PERM)
    def _():
        pltpu.sync_copy(pos_hbm.at[wid], pos_v)

        def body(j, carry):
            idx = pos_v[pl.ds(j * 16, 16)]
            val = lax.iota(jnp.int32, 16) + j * 16
            plsc.store_scatter(tab_v, [idx], val)
            return carry

        lax.fori_loop(0, S // 16, body, 0)
        pltpu.sync_copy(tab_v, out_hbm.at[wid])


def _sc_invert_perm(pos):
    fn = functools.partial(
        pl.kernel,
        mesh=plsc.VectorSubcoreMesh(core_axis_name="c", subcore_axis_name="s"),
        out_type=jax.ShapeDtypeStruct((_N_PERM, S), jnp.int32),
        scratch_types=[
            pltpu.VMEM((S,), jnp.int32),
            pltpu.VMEM((S,), jnp.int32),
        ],
        compiler_params=pltpu.CompilerParams(needs_layout_passes=False),
    )(_sc_inv_body)
    return fn(pos)


def kernel(x, ln1_w, ln1_b, w_qkv, b_qkv, w_out, b_out, ln2_w, ln2_b,
           w1, b1, w2, b2, alpha, beta):
    x2 = x[0]
    hd_ids = jnp.arange(NUM_HEADS, dtype=jnp.int32)
    kv_parts = jnp.stack([NUM_HEADS + hd_ids, 2 * NUM_HEADS + hd_ids],
                         axis=1).reshape(-1)
    parts = jnp.concatenate([hd_ids, kv_parts])
    perm = (parts[:, None] * HEAD_DIM
            + jnp.arange(HEAD_DIM, dtype=jnp.int32)[None, :]).reshape(-1)
    qkv = _ln_qkv(x2, ln1_w, ln1_b, w_qkv[perm], b_qkv[perm])
    qkv_flat = qkv.reshape(S * NPART, HEAD_DIM)
    qkv_flat128 = qkv.reshape(S * NPART // 2, 2 * HEAD_DIM)

    q = qkv[:, :HIDDEN_DIM].reshape(
        S, NUM_HEADS, HEAD_DIM).transpose(1, 0, 2)
    k = qkv[:, HIDDEN_DIM:].reshape(S, NUM_HEADS, 2, HEAD_DIM)[:, :, 0, :]
    k = k.transpose(1, 0, 2)

    q_norms = jnp.linalg.norm(q, axis=-1, keepdims=True)
    k_norms = jnp.linalg.norm(k, axis=-1, keepdims=True)
    MX = jnp.max(q_norms, axis=1, keepdims=True)
    MY = jnp.max(k_norms, axis=1, keepdims=True)
    q_ext = jnp.sqrt(jnp.maximum(MX ** 2 + MY ** 2 - q_norms ** 2, 0.0))
    k_ext = jnp.sqrt(jnp.maximum(MX ** 2 + MY ** 2 - k_norms ** 2, 0.0))
    Qh = jnp.concatenate([q, q_ext, jnp.zeros_like(q_ext)], axis=-1)
    Kh = jnp.concatenate([k, jnp.zeros_like(k_ext), k_ext], axis=-1)
    q_proj = jax.lax.stop_gradient(Qh @ alpha + beta)
    k_proj = jax.lax.stop_gradient(Kh @ alpha + beta)

    keys = jnp.stack([q_proj.transpose(2, 0, 1), k_proj.transpose(2, 0, 1)])
    kb = jax.lax.bitcast_convert_type(keys, jnp.int32)
    kb = kb ^ ((kb >> 31) & jnp.int32(0x7FFFFFFF))
    positions = jnp.argsort(kb, axis=-1)
    q_positions, k_positions = positions[0], positions[1]

    q_rev = _sc_invert_perm(q_positions.reshape(N_HASHES * NUM_HEADS, S))
    q_rev = q_rev.reshape(q_positions.shape)

    head_part = jnp.arange(NUM_HEADS, dtype=jnp.int32)[None, :, None]
    q_idx = q_positions * NPART + head_part
    kv_idx = k_positions * (NPART // 2) + NUM_HEADS // 2 + head_part
    s_q = jnp.take(qkv_flat, q_idx.reshape(-1), axis=0).reshape(-1, Q_CLUSTER, HEAD_DIM)
    s_kv = jnp.take(qkv_flat128, kv_idx.reshape(-1), axis=0).reshape(
        -1, K_CLUSTER, 2 * HEAD_DIM)

    bo, lse = _cluster_attn(s_q, s_kv)

    g = jnp.arange(N_HASHES * NUM_HEADS, dtype=jnp.int32)
    o_idx = g[None, :] * S + q_rev.reshape(N_HASHES * NUM_HEADS, S).T
    o_tm = jnp.take(bo.reshape(-1, HEAD_DIM), o_idx.reshape(-1),
                    axis=0).reshape(S, N_HASHES * HIDDEN_DIM)
    slog = jnp.take(lse.reshape(-1), o_idx.reshape(-1), axis=0).reshape(
        S, N_HASHES, NUM_HEADS)
    mx = jnp.max(slog, axis=1, keepdims=True)
    w = jnp.exp(slog - mx)
    probs_tm = (w / jnp.sum(w, axis=1, keepdims=True)).reshape(
        S, N_HASHES * NUM_HEADS)

    out = _fused_tail(o_tm, probs_tm, x2, w_out, b_out, ln2_w, ln2_b,
                      w1, b1, w2, b2)
    return out[None]


# speedup vs baseline: 1.0533x; 1.0053x over previous
import functools

import jax
import jax.numpy as jnp
from jax import lax
from jax.experimental import pallas as pl
from jax.experimental.pallas import tpu as pltpu
from jax.experimental.pallas import tpu_sc as plsc

NUM_HEADS = 12
HIDDEN_DIM = 768
MLP_DIM = 3072
N_HASHES = 2
Q_CLUSTER = 128
K_CLUSTER = 128
HEAD_DIM = HIDDEN_DIM // NUM_HEADS
S = 4096
NPART = 3 * NUM_HEADS

ROW_BLK = 512
ATTN_BLK = 16
MLP_BLK = 768


def _ln(x, w, b, eps=1e-6):
    mu = jnp.mean(x, axis=-1, keepdims=True)
    var = jnp.mean((x - mu) ** 2, axis=-1, keepdims=True)
    return (x - mu) / jnp.sqrt(var + eps) * w + b


def _bdot(a, b, dims):
    return jax.lax.dot_general(a.astype(jnp.bfloat16), b.astype(jnp.bfloat16),
                               dims, preferred_element_type=jnp.float32)


def _k1_body(x_ref, w_ref, lw_ref, lb_ref, bq_ref, out_ref):
    xn = _ln(x_ref[...], lw_ref[...], lb_ref[...])
    acc = _bdot(xn, w_ref[...], (((1,), (1,)), ((), ())))
    col = jax.lax.broadcasted_iota(jnp.int32, (1, 3 * HIDDEN_DIM), 1)
    scale = jnp.where(col < HIDDEN_DIM, HEAD_DIM ** -0.5, 1.0)
    out_ref[...] = acc * scale + bq_ref[...] * scale


def _ln_qkv(x, ln1_w, ln1_b, w_qkv, b_qkv):
    return pl.pallas_call(
        _k1_body,
        grid=(S // ROW_BLK,),
        in_specs=[
            pl.BlockSpec((ROW_BLK, HIDDEN_DIM), lambda r: (r, 0)),
            pl.BlockSpec((3 * HIDDEN_DIM, HIDDEN_DIM), lambda r: (0, 0)),
            pl.BlockSpec((1, HIDDEN_DIM), lambda r: (0, 0)),
            pl.BlockSpec((1, HIDDEN_DIM), lambda r: (0, 0)),
            pl.BlockSpec((1, 3 * HIDDEN_DIM), lambda r: (0, 0)),
        ],
        out_specs=pl.BlockSpec((ROW_BLK, 3 * HIDDEN_DIM), lambda r: (r, 0)),
        out_shape=jax.ShapeDtypeStruct((S, 3 * HIDDEN_DIM), jnp.float32),
    )(x, w_qkv, ln1_w.reshape(1, -1), ln1_b.reshape(1, -1), b_qkv.reshape(1, -1))


def _k2_body(q_ref, kv_ref, bo_ref, lse_ref):
    for i in range(ATTN_BLK):
        q = q_ref[i]
        kv = kv_ref[i]
        k = kv[:, :HEAD_DIM]
        v = kv[:, HEAD_DIM:]
        inner = _bdot(q, k, (((1,), (1,)), ((), ())))
        m = jnp.max(inner, axis=-1, keepdims=True)
        e = jnp.exp(inner - m)
        s = jnp.sum(e, axis=-1, keepdims=True)
        bo = _bdot(e / s, v, (((1,), (0,)), ((), ())))
        bo_ref[i] = bo
        lse_ref[i] = (m + jnp.log(s))[:, 0]


def _cluster_attn(s_q, s_kv):
    n_c = s_q.shape[0]
    return pl.pallas_call(
        _k2_body,
        grid=(n_c // ATTN_BLK,),
        in_specs=[
            pl.BlockSpec((ATTN_BLK, Q_CLUSTER, HEAD_DIM), lambda c: (c, 0, 0)),
            pl.BlockSpec((ATTN_BLK, K_CLUSTER, 2 * HEAD_DIM), lambda c: (c, 0, 0)),
        ],
        out_specs=[
            pl.BlockSpec((ATTN_BLK, Q_CLUSTER, HEAD_DIM), lambda c: (c, 0, 0)),
            pl.BlockSpec((ATTN_BLK, Q_CLUSTER), lambda c: (c, 0)),
        ],
        out_shape=[
            jax.ShapeDtypeStruct((n_c, Q_CLUSTER, HEAD_DIM), jnp.float32),
            jax.ShapeDtypeStruct((n_c, Q_CLUSTER), jnp.float32),
        ],
    )(s_q, s_kv)


def _k3_body(o_ref, p_ref, x_ref, wo_ref, bo_ref, l2w_ref, l2b_ref,
             w1_ref, b1_ref, w2_ref, b2_ref, out_ref, h_s, y_s):
    m = pl.program_id(1)

    @pl.when(m == 0)
    def _():
        o = o_ref[...]
        p = p_ref[...]
        row = jax.lax.broadcasted_iota(jnp.int32, (NUM_HEADS, HIDDEN_DIM), 0)
        col = jax.lax.broadcasted_iota(jnp.int32, (NUM_HEADS, HIDDEN_DIM), 1)
        E = jnp.where(col // HEAD_DIM == row, 1.0, 0.0)
        pe1 = jax.lax.dot_general(p[:, :NUM_HEADS], E, (((1,), (0,)), ((), ())),
                                  preferred_element_type=jnp.float32)
        pe2 = jax.lax.dot_general(p[:, NUM_HEADS:], E, (((1,), (0,)), ((), ())),
                                  preferred_element_type=jnp.float32)
        attn = o[:, :HIDDEN_DIM] * pe1 + o[:, HIDDEN_DIM:] * pe2
        h = _bdot(attn, wo_ref[...], (((1,), (1,)), ((), ())))
        h = h + bo_ref[...] + x_ref[...]
        h_s[...] = h
        y_s[...] = _ln(h, l2w_ref[...], l2b_ref[...])

    z = _bdot(y_s[...], w1_ref[...], (((1,), (1,)), ((), ())))
    z = z + b1_ref[...]
    z = 0.5 * z * (1.0 + jax.lax.erf(z * (2.0 ** -0.5)))
    part = _bdot(z, w2_ref[...], (((1,), (1,)), ((), ())))

    @pl.when(m == 0)
    def _():
        out_ref[...] = h_s[...] + b2_ref[...] + part

    @pl.when(m != 0)
    def _():
        out_ref[...] = out_ref[...] + part


def _fused_tail(o_tm, probs_tm, x, w_out, b_out, ln2_w, ln2_b, w1, b1, w2, b2):
    n_m = MLP_DIM // MLP_BLK
    return pl.pallas_call(
        _k3_body,
        grid=(S // ROW_BLK, n_m),
        in_specs=[
            pl.BlockSpec((ROW_BLK, N_HASHES * HIDDEN_DIM), lambda r, m: (r, 0)),
            pl.BlockSpec((ROW_BLK, N_HASHES * NUM_HEADS), lambda r, m: (r, 0)),
            pl.BlockSpec((ROW_BLK, HIDDEN_DIM), lambda r, m: (r, 0)),
            pl.BlockSpec((HIDDEN_DIM, HIDDEN_DIM), lambda r, m: (0, 0)),
            pl.BlockSpec((1, HIDDEN_DIM), lambda r, m: (0, 0)),
            pl.BlockSpec((1, HIDDEN_DIM), lambda r, m: (0, 0)),
            pl.BlockSpec((1, HIDDEN_DIM), lambda r, m: (0, 0)),
            pl.BlockSpec((MLP_BLK, HIDDEN_DIM), lambda r, m: (m, 0)),
            pl.BlockSpec((1, MLP_BLK), lambda r, m: (0, m)),
            pl.BlockSpec((HIDDEN_DIM, MLP_BLK), lambda r, m: (0, m)),
            pl.BlockSpec((1, HIDDEN_DIM), lambda r, m: (0, 0)),
        ],
        out_specs=pl.BlockSpec((ROW_BLK, HIDDEN_DIM), lambda r, m: (r, 0)),
        out_shape=jax.ShapeDtypeStruct((S, HIDDEN_DIM), jnp.float32),
        scratch_shapes=[
            pltpu.VMEM((ROW_BLK, HIDDEN_DIM), jnp.float32),
            pltpu.VMEM((ROW_BLK, HIDDEN_DIM), jnp.float32),
        ],
    )(o_tm, probs_tm, x, w_out, b_out.reshape(1, -1), ln2_w.reshape(1, -1),
      ln2_b.reshape(1, -1), w1, b1.reshape(1, -1), w2, b2.reshape(1, -1))


_N_PERM = N_HASHES * NUM_HEADS


def _sc_inv_body(pos_hbm, out_hbm, pos_v, tab_v):
    wid = lax.axis_index("s") * 2 + lax.axis_index("c")

    @pl.when(wid < _N_PERM)
    def _():
        pltpu.sync_copy(pos_hbm.at[wid], pos_v)

        def body(j, carry):
            idx = pos_v[pl.ds(j * 16, 16)]
            val = lax.iota(jnp.int32, 16) + j * 16
            plsc.store_scatter(tab_v, [idx], val)
            return carry

        lax.fori_loop(0, S // 16, body, 0)
        pltpu.sync_copy(tab_v, out_hbm.at[wid])


def _sc_invert_perm(pos):
    fn = functools.partial(
        pl.kernel,
        mesh=plsc.VectorSubcoreMesh(core_axis_name="c", subcore_axis_name="s"),
        out_type=jax.ShapeDtypeStruct((_N_PERM, S), jnp.int32),
        scratch_types=[
            pltpu.VMEM((S,), jnp.int32),
            pltpu.VMEM((S,), jnp.int32),
        ],
        compiler_params=pltpu.CompilerParams(needs_layout_passes=False),
    )(_sc_inv_body)
    return fn(pos)


def kernel(x, ln1_w, ln1_b, w_qkv, b_qkv, w_out, b_out, ln2_w, ln2_b,
           w1, b1, w2, b2, alpha, beta):
    x2 = x[0]
    hd_ids = jnp.arange(NUM_HEADS, dtype=jnp.int32)
    kv_parts = jnp.stack([NUM_HEADS + hd_ids, 2 * NUM_HEADS + hd_ids],
                         axis=1).reshape(-1)
    parts = jnp.concatenate([hd_ids, kv_parts])
    perm = (parts[:, None] * HEAD_DIM
            + jnp.arange(HEAD_DIM, dtype=jnp.int32)[None, :]).reshape(-1)
    qkv = _ln_qkv(x2, ln1_w, ln1_b, w_qkv[perm], b_qkv[perm])
    qkv_flat = qkv.reshape(S * NPART, HEAD_DIM)
    qkv_flat128 = qkv.reshape(S * NPART // 2, 2 * HEAD_DIM)

    q = qkv[:, :HIDDEN_DIM].reshape(
        S, NUM_HEADS, HEAD_DIM).transpose(1, 0, 2)
    k = qkv[:, HIDDEN_DIM:].reshape(S, NUM_HEADS, 2, HEAD_DIM)[:, :, 0, :]
    k = k.transpose(1, 0, 2)

    q_norms = jnp.linalg.norm(q, axis=-1, keepdims=True)
    k_norms = jnp.linalg.norm(k, axis=-1, keepdims=True)
    MX = jnp.max(q_norms, axis=1, keepdims=True)
    MY = jnp.max(k_norms, axis=1, keepdims=True)
    q_ext = jnp.sqrt(jnp.maximum(MX ** 2 + MY ** 2 - q_norms ** 2, 0.0))
    k_ext = jnp.sqrt(jnp.maximum(MX ** 2 + MY ** 2 - k_norms ** 2, 0.0))
    Qh = jnp.concatenate([q, q_ext, jnp.zeros_like(q_ext)], axis=-1)
    Kh = jnp.concatenate([k, jnp.zeros_like(k_ext), k_ext], axis=-1)
    q_proj = jax.lax.stop_gradient(Qh @ alpha + beta)
    k_proj = jax.lax.stop_gradient(Kh @ alpha + beta)

    keys = jnp.stack([q_proj.transpose(2, 0, 1), k_proj.transpose(2, 0, 1)])
    kb = jax.lax.bitcast_convert_type(keys, jnp.int32)
    kb = kb ^ ((kb >> 31) & jnp.int32(0x7FFFFFFF))
    positions = jnp.argsort(kb, axis=-1)
    q_positions, k_positions = positions[0], positions[1]

    q_rev = _sc_invert_perm(q_positions.reshape(N_HASHES * NUM_HEADS, S))
    q_rev = q_rev.reshape(q_positions.shape)

    head_part = jnp.arange(NUM_HEADS, dtype=jnp.int32)[None, :, None]
    q_idx = q_positions * NPART + head_part
    kv_idx = k_positions * (NPART // 2) + NUM_HEADS // 2 + head_part
    s_q = jnp.take(qkv_flat, q_idx.reshape(-1), axis=0).reshape(-1, Q_CLUSTER, HEAD_DIM)
    s_kv = jnp.take(qkv_flat128, kv_idx.reshape(-1), axis=0).reshape(
        -1, K_CLUSTER, 2 * HEAD_DIM)

    bo, lse = _cluster_attn(s_q, s_kv)

    g = jnp.arange(N_HASHES * NUM_HEADS, dtype=jnp.int32)
    o_idx = g[None, :] * S + q_rev.reshape(N_HASHES * NUM_HEADS, S).T
    o_tm = jnp.take(bo.reshape(-1, HEAD_DIM), o_idx.reshape(-1),
                    axis=0).reshape(S, N_HASHES * HIDDEN_DIM)
    slog = jnp.take(lse.reshape(-1), o_idx.reshape(-1), axis=0).reshape(
        S, N_HASHES, NUM_HEADS)
    mx = jnp.max(slog, axis=1, keepdims=True)
    w = jnp.exp(slog - mx)
    probs_tm = (w / jnp.sum(w, axis=1, keepdims=True)).reshape(
        S, N_HASHES * NUM_HEADS)

    out = _fused_tail(o_tm, probs_tm, x2, w_out, b_out, ln2_w, ln2_b,
                      w1, b1, w2, b2)
    return out[None]
